# baseline (speedup 1.0000x reference)
import functools

import jax
import jax.numpy as jnp
from jax import lax
from jax.experimental import pallas as pl
from jax.experimental.pallas import tpu as pltpu
from jax.experimental.pallas import tpu_sc as plsc

N_NODES = 10000
N_EDGES = 320000
D_FEAT = 128
D_DEG = 16

NUM_CORES = 2
NUM_SUBCORES = 16
NUM_WORKERS = NUM_CORES * NUM_SUBCORES
EDGES_PER_WORKER = N_EDGES // NUM_WORKERS
CHUNK = 80
NUM_CHUNKS = EDGES_PER_WORKER // CHUNK
ROWS_PER_TILE = N_NODES // NUM_SUBCORES


def _sc_body(feat_hbm, src_hbm, dst_hbm, zf_hbm, zd_hbm, outf_hbm, outd_hbm,
             accf_sh, accd_sh, src_v, dst_v, rows0, rows1, ones_v,
             g0, g1, sd, sf, zs):
    cid = lax.axis_index("c")
    sid = lax.axis_index("s")
    wid = cid * NUM_SUBCORES + sid
    row0 = sid * ROWS_PER_TILE

    pltpu.async_copy(zf_hbm, accf_sh.at[pl.ds(row0, ROWS_PER_TILE)], zs)
    pltpu.async_copy(zd_hbm, accd_sh.at[pl.ds(row0, ROWS_PER_TILE)], zs)

    ovec = jnp.ones((2, 16), jnp.bfloat16)

    def _orow(i, _):
        ones_v[pl.ds(2 * i, 2), :] = ovec
        return 0

    lax.fori_loop(0, CHUNK // 2, _orow, 0)

    pltpu.sync_copy(src_hbm.at[wid], src_v)
    pltpu.sync_copy(dst_hbm.at[wid], dst_v)

    pltpu.make_async_copy(zf_hbm, accf_sh.at[pl.ds(row0, ROWS_PER_TILE)], zs).wait()
    pltpu.make_async_copy(zd_hbm, accd_sh.at[pl.ds(row0, ROWS_PER_TILE)], zs).wait()

    plsc.subcore_barrier()

    pltpu.async_copy(feat_hbm.at[src_v.at[0]], rows0, g0)
    pltpu.async_copy(feat_hbm.at[src_v.at[1]], rows1, g1)

    def _step(i, _):
        j = 2 * i
        pltpu.make_async_copy(feat_hbm.at[src_v.at[j]], rows0, g0).wait()
        pltpu.async_copy(rows0, accf_sh.at[dst_v.at[j]], sf, add=True)

        @pl.when(i > 0)
        def _():
            pltpu.make_async_copy(ones_v, accd_sh.at[dst_v.at[0]], sd).wait()

        pltpu.async_copy(ones_v, accd_sh.at[dst_v.at[j]], sd, add=True)
        pltpu.make_async_copy(rows0, accf_sh.at[dst_v.at[j]], sf).wait()

        @pl.when(i < NUM_CHUNKS // 2 - 1)
        def _():
            pltpu.async_copy(feat_hbm.at[src_v.at[j + 2]], rows0, g0)

        pltpu.make_async_copy(feat_hbm.at[src_v.at[j + 1]], rows1, g1).wait()
        pltpu.async_copy(rows1, accf_sh.at[dst_v.at[j + 1]], sf, add=True)
        pltpu.make_async_copy(ones_v, accd_sh.at[dst_v.at[0]], sd).wait()
        pltpu.async_copy(ones_v, accd_sh.at[dst_v.at[j + 1]], sd, add=True)
        pltpu.make_async_copy(rows1, accf_sh.at[dst_v.at[j + 1]], sf).wait()

        @pl.when(i < NUM_CHUNKS // 2 - 1)
        def _():
            pltpu.async_copy(feat_hbm.at[src_v.at[j + 3]], rows1, g1)

        return 0

    lax.fori_loop(0, NUM_CHUNKS // 2, _step, 0)

    if NUM_CHUNKS % 2:
        last = NUM_CHUNKS - 1
        pltpu.async_copy(feat_hbm.at[src_v.at[last]], rows0, g0)
        pltpu.make_async_copy(feat_hbm.at[src_v.at[last]], rows0, g0).wait()
        pltpu.async_copy(rows0, accf_sh.at[dst_v.at[last]], sf, add=True)
        pltpu.make_async_copy(ones_v, accd_sh.at[dst_v.at[0]], sd).wait()
        pltpu.async_copy(ones_v, accd_sh.at[dst_v.at[last]], sd, add=True)
        pltpu.make_async_copy(rows0, accf_sh.at[dst_v.at[last]], sf).wait()

    pltpu.make_async_copy(ones_v, accd_sh.at[dst_v.at[0]], sd).wait()

    plsc.subcore_barrier()

    out_base = cid * N_NODES + sid * ROWS_PER_TILE
    pltpu.sync_copy(accf_sh.at[pl.ds(row0, ROWS_PER_TILE)],
                    outf_hbm.at[pl.ds(out_base, ROWS_PER_TILE)])
    pltpu.sync_copy(accd_sh.at[pl.ds(row0, ROWS_PER_TILE)],
                    outd_hbm.at[pl.ds(out_base, ROWS_PER_TILE)])


@functools.lru_cache(maxsize=1)
def _sc_agg():
    return functools.partial(
        pl.kernel,
        out_type=(
            jax.ShapeDtypeStruct((NUM_CORES * N_NODES, D_FEAT), jnp.float32),
            jax.ShapeDtypeStruct((NUM_CORES * N_NODES, D_DEG), jnp.bfloat16),
        ),
        mesh=plsc.VectorSubcoreMesh(core_axis_name="c", subcore_axis_name="s"),
        scratch_types=[
            pltpu.VMEM_SHARED((N_NODES, D_FEAT), jnp.float32),
            pltpu.VMEM_SHARED((N_NODES, D_DEG), jnp.bfloat16),
            pltpu.VMEM((NUM_CHUNKS, CHUNK), jnp.int32),
            pltpu.VMEM((NUM_CHUNKS, CHUNK), jnp.int32),
            pltpu.VMEM((CHUNK, D_FEAT), jnp.float32),
            pltpu.VMEM((CHUNK, D_FEAT), jnp.float32),
            pltpu.VMEM((CHUNK, D_DEG), jnp.bfloat16),
            pltpu.SemaphoreType.DMA,
            pltpu.SemaphoreType.DMA,
            pltpu.SemaphoreType.DMA,
            pltpu.SemaphoreType.DMA,
            pltpu.SemaphoreType.DMA,
        ],
        compiler_params=pltpu.CompilerParams(use_tc_tiling_on_sc=False),
    )(_sc_body)


def _tc_body(pf_ref, pd_ref, f_ref, w_ref, b_ref, o_ref):
    feat_sum = pf_ref[0] + pf_ref[1]
    deg = (pd_ref[0, :, :1].astype(jnp.float32)
           + pd_ref[1, :, :1].astype(jnp.float32))
    mean = feat_sum / jnp.maximum(deg, 1.0)
    h = jnp.where(deg > 0.0, mean, f_ref[...])
    y = lax.dot_general(h, w_ref[...], (((1,), (1,)), ((), ())),
                        preferred_element_type=jnp.float32)
    o_ref[...] = jnp.maximum(y + b_ref[...], 0.0)


_BR = 1000


def _tc_finish(pfeat, pdeg, features, wt, b2):
    grid = (N_NODES // _BR,)
    return pl.pallas_call(
        _tc_body,
        grid=grid,
        in_specs=[
            pl.BlockSpec((NUM_CORES, _BR, D_FEAT), lambda i: (0, i, 0)),
            pl.BlockSpec((NUM_CORES, _BR, D_DEG), lambda i: (0, i, 0)),
            pl.BlockSpec((_BR, D_FEAT), lambda i: (i, 0)),
            pl.BlockSpec((D_FEAT, D_FEAT), lambda i: (0, 0)),
            pl.BlockSpec((1, D_FEAT), lambda i: (0, 0)),
        ],
        out_specs=pl.BlockSpec((_BR, D_FEAT), lambda i: (i, 0)),
        out_shape=jax.ShapeDtypeStruct((N_NODES, D_FEAT), jnp.float32),
    )(pfeat, pdeg, features, wt, b2)


def kernel(features, edge_index, W, b):
    src = edge_index[0].astype(jnp.int32).reshape(NUM_WORKERS, NUM_CHUNKS, CHUNK)
    dst = edge_index[1].astype(jnp.int32).reshape(NUM_WORKERS, NUM_CHUNKS, CHUNK)
    zf = jnp.zeros((ROWS_PER_TILE, D_FEAT), jnp.float32)
    zd = jnp.zeros((ROWS_PER_TILE, D_DEG), jnp.bfloat16)
    pfeat, pdeg = _sc_agg()(features, src, dst, zf, zd)
    pfeat = pfeat.reshape(NUM_CORES, N_NODES, D_FEAT)
    pdeg = pdeg.reshape(NUM_CORES, N_NODES, D_DEG)
    return _tc_finish(pfeat, pdeg, features, W, b.reshape(1, D_FEAT))
